# baseline (speedup 1.0000x reference)
import functools

import jax
import jax.numpy as jnp
from jax import lax
from jax.experimental import pallas as pl
from jax.experimental.pallas import tpu as pltpu
from jax.experimental.pallas import tpu_sc as plsc

VOCAB = 1000000
N = 819200
DIM = 32
NUM_CORES = 2
NUM_SUBCORES = 16
NW = NUM_CORES * NUM_SUBCORES
B_PER_W = N // NW
CHUNK = 1024
NCHUNK = B_PER_W // CHUNK
NGRP = N // 128
CGRP = CHUNK // 128


def _sc_gather(inputs, table):
    mesh = plsc.VectorSubcoreMesh(core_axis_name="c", subcore_axis_name="s")

    scratch = [
        pltpu.VMEM((CHUNK,), jnp.int32),
        pltpu.VMEM((CHUNK, DIM), jnp.float32),
        pltpu.VMEM((CHUNK * DIM,), jnp.float32),
        pltpu.SemaphoreType.DMA,
    ]

    @functools.partial(
        pl.kernel,
        mesh=mesh,
        out_type=jax.ShapeDtypeStruct((N * DIM,), jnp.float32),
        scratch_types=scratch,
        compiler_params=pltpu.CompilerParams(
            use_tc_tiling_on_sc=False, needs_layout_passes=False),
    )
    def k(idx_hbm, t_hbm, out_hbm, idx_v, rows_v, nat_v, gsem):
        wid = lax.axis_index("s") * NUM_CORES + lax.axis_index("c")
        base = wid * B_PER_W
        lane = lax.iota(jnp.int32, 16)

        def chunk_body(i, _):
            off = base + i * CHUNK
            pltpu.sync_copy(idx_hbm.at[pl.ds(off, CHUNK)], idx_v)
            pltpu.async_copy(t_hbm.at[idx_v], rows_v, gsem).wait()

            def tr_body(s, _):
                j16 = s * 16 + lane
                g = lax.shift_right_logical(j16, 7)
                ii = j16 & 127
                dst0 = g * 1024 + ii
                for c in range(DIM):
                    rt = c // 8
                    r = c % 8
                    val = plsc.load_gather(
                        rows_v, [j16, jnp.full((16,), c, jnp.int32)])
                    plsc.store_scatter(
                        nat_v, [dst0 + (rt * CGRP * 1024 + r * 128)], val)
                return ()
            lax.fori_loop(0, CHUNK // 16, tr_body, ())

            g0 = off // 128
            for rt in range(4):
                seg = CGRP * 8 * 128
                pltpu.sync_copy(
                    nat_v.at[pl.ds(rt * seg, seg)],
                    out_hbm.at[pl.ds((rt * NGRP + g0) * 1024, seg)],
                )
            return ()

        lax.fori_loop(0, NCHUNK, chunk_body, ())

    return k(inputs, table)


def kernel(inputs, table):
    out_flat = _sc_gather(inputs, table)
    out = out_flat.reshape(4, NGRP, 8, 128).transpose(1, 3, 0, 2).reshape(N, DIM)
    return out
